# Initial kernel scaffold; baseline (speedup 1.0000x reference)
import jax
import jax.numpy as jnp
from jax import lax
from jax.experimental import pallas as pl
from jax.experimental.pallas import tpu as pltpu
from jax.experimental.pallas import tpu_sc as plsc

N = 10000
E = 320000
D = 128

NC = 2
NS = 16
NW = NC * NS
LANES = 16

EPW = E // NW
B = 512
NB = E // B
GMAX = (NB + NW - 1) // NW
RPS = N // NS

_vec_mesh = plsc.VectorSubcoreMesh(
    core_axis_name="c", subcore_axis_name="s", num_cores=NC, num_subcores=NS
)


def _sc_degrees(src, dst):
  @pl.kernel(
      out_type=(
          jax.ShapeDtypeStruct((NW, N), jnp.float32),
          jax.ShapeDtypeStruct((NW, N), jnp.float32),
      ),
      mesh=_vec_mesh,
      scratch_types=[
          pltpu.VMEM((EPW,), jnp.int32),
          pltpu.VMEM((EPW,), jnp.int32),
          pltpu.VMEM((N,), jnp.float32),
          pltpu.VMEM((N,), jnp.float32),
      ],
  )
  def deg_kernel(src_hbm, dst_hbm, dout_hbm, din_hbm, sv, dv, aout, ain):
    cid = lax.axis_index("c")
    sid = lax.axis_index("s")
    wid = sid * NC + cid

    pltpu.sync_copy(src_hbm.at[pl.ds(wid * EPW, EPW)], sv)
    pltpu.sync_copy(dst_hbm.at[pl.ds(wid * EPW, EPW)], dv)

    zeros = jnp.zeros((LANES,), jnp.float32)
    ones = jnp.ones((LANES,), jnp.float32)

    @pl.loop(0, N, step=LANES)
    def _(i):
      aout[pl.ds(i, LANES)] = zeros
      ain[pl.ds(i, LANES)] = zeros

    @pl.loop(0, EPW, step=LANES)
    def _(i):
      s = sv[pl.ds(i, LANES)]
      d = dv[pl.ds(i, LANES)]
      plsc.addupdate_scatter(aout, [s], ones)
      plsc.addupdate_scatter(ain, [d], ones)

    pltpu.sync_copy(aout, dout_hbm.at[wid])
    pltpu.sync_copy(ain, din_hbm.at[wid])

  return deg_kernel(src, dst)


def _sc_segsum_rows(y, srcb, dstb):
  @pl.kernel(
      out_type=jax.ShapeDtypeStruct((NC, N, D), jnp.float32),
      mesh=_vec_mesh,
      scratch_types=[
          pltpu.VMEM((1, B), jnp.int32),
          pltpu.VMEM((1, B), jnp.int32),
          pltpu.VMEM((B, D), jnp.float32),
          pltpu.VMEM_SHARED((N, D), jnp.float32),
      ],
  )
  def seg_kernel(y_hbm, srcb_hbm, dstb_hbm, zero_hbm, out_hbm, si, di, rows,
                 acc):
    cid = lax.axis_index("c")
    sid = lax.axis_index("s")
    wid = sid * NC + cid

    pltpu.sync_copy(zero_hbm.at[pl.ds(sid * RPS, RPS)],
                    acc.at[pl.ds(sid * RPS, RPS)])
    plsc.subcore_barrier()

    @pl.loop(0, GMAX)
    def _(g):
      b = g * NW + wid

      @pl.when(b < NB)
      def _():
        pltpu.sync_copy(srcb_hbm.at[pl.ds(b, 1)], si)
        pltpu.sync_copy(dstb_hbm.at[pl.ds(b, 1)], di)
        pltpu.sync_copy(y_hbm.at[si.at[0]], rows)
        pltpu.sync_copy(rows, acc.at[di.at[0]], add=True)

    plsc.subcore_barrier()
    pltpu.sync_copy(acc.at[pl.ds(sid * RPS, RPS)],
                    out_hbm.at[cid].at[pl.ds(sid * RPS, RPS)])

  zero = jnp.zeros((N, D), jnp.float32)
  return seg_kernel(y, srcb, dstb, zero)


def _sc_segsum_scalar(z, src, dst):
  @pl.kernel(
      out_type=jax.ShapeDtypeStruct((NW, N), jnp.float32),
      mesh=_vec_mesh,
      scratch_types=[
          pltpu.VMEM((N,), jnp.float32),
          pltpu.VMEM((EPW,), jnp.int32),
          pltpu.VMEM((EPW,), jnp.int32),
          pltpu.VMEM((N,), jnp.float32),
      ],
  )
  def seg2_kernel(z_hbm, src_hbm, dst_hbm, out_hbm, zv, sv, dv, acc):
    cid = lax.axis_index("c")
    sid = lax.axis_index("s")
    wid = sid * NC + cid

    pltpu.sync_copy(z_hbm, zv)
    pltpu.sync_copy(src_hbm.at[pl.ds(wid * EPW, EPW)], sv)
    pltpu.sync_copy(dst_hbm.at[pl.ds(wid * EPW, EPW)], dv)

    zeros = jnp.zeros((LANES,), jnp.float32)

    @pl.loop(0, N, step=LANES)
    def _(i):
      acc[pl.ds(i, LANES)] = zeros

    @pl.loop(0, EPW, step=LANES)
    def _(i):
      s = sv[pl.ds(i, LANES)]
      d = dv[pl.ds(i, LANES)]
      vals = plsc.load_gather(zv, [s])
      plsc.addupdate_scatter(acc, [d], vals)

    pltpu.sync_copy(acc, out_hbm.at[wid])

  return seg2_kernel(z, src, dst)


R = 1000


def _tc_mm1(doutp, dinp, x, w1):
  def body(doutp_ref, dinp_ref, x_ref, w1_ref, y_ref, no_ref, ni_ref):
    no = lax.rsqrt(jnp.clip(jnp.sum(doutp_ref[...], axis=0), 1.0, None))
    ni = lax.rsqrt(jnp.clip(jnp.sum(dinp_ref[...], axis=0), 1.0, None))
    no = no[:, None]
    ni = ni[:, None]
    y_ref[...] = (
        jnp.dot(x_ref[...], w1_ref[...], preferred_element_type=jnp.float32)
        * no)
    no_ref[...] = no
    ni_ref[...] = ni

  return pl.pallas_call(
      body,
      grid=(N // R,),
      in_specs=[
          pl.BlockSpec((NW, R), lambda i: (0, i)),
          pl.BlockSpec((NW, R), lambda i: (0, i)),
          pl.BlockSpec((R, D), lambda i: (i, 0)),
          pl.BlockSpec((D, D), lambda i: (0, 0)),
      ],
      out_specs=[
          pl.BlockSpec((R, D), lambda i: (i, 0)),
          pl.BlockSpec((R, 1), lambda i: (i, 0)),
          pl.BlockSpec((R, 1), lambda i: (i, 0)),
      ],
      out_shape=[
          jax.ShapeDtypeStruct((N, D), jnp.float32),
          jax.ShapeDtypeStruct((N, 1), jnp.float32),
          jax.ShapeDtypeStruct((N, 1), jnp.float32),
      ],
  )(doutp, dinp, x, w1)


def _tc_mm2(aggp, nin, nout, b1, w2t):
  def body(aggp_ref, ni_ref, no_ref, b1_ref, w2_ref, z_ref):
    agg = aggp_ref[0] + aggp_ref[1]
    h = jnp.maximum(agg * ni_ref[...] + b1_ref[...], 0.0)
    z_ref[...] = jnp.sum(h * w2_ref[...], axis=1, keepdims=True) * no_ref[...]

  return pl.pallas_call(
      body,
      grid=(N // R,),
      in_specs=[
          pl.BlockSpec((NC, R, D), lambda i: (0, i, 0)),
          pl.BlockSpec((R, 1), lambda i: (i, 0)),
          pl.BlockSpec((R, 1), lambda i: (i, 0)),
          pl.BlockSpec((1, D), lambda i: (0, 0)),
          pl.BlockSpec((1, D), lambda i: (0, 0)),
      ],
      out_specs=pl.BlockSpec((R, 1), lambda i: (i, 0)),
      out_shape=jax.ShapeDtypeStruct((N, 1), jnp.float32),
  )(aggp, nin, nout, b1, w2t)


def _tc_out(a2p, nin, b2):
  def body(a2p_ref, ni_ref, b2_ref, o_ref):
    s = jnp.sum(a2p_ref[...], axis=0)
    ni = ni_ref[...][:, 0]
    o_ref[...] = jax.nn.sigmoid(s * ni + b2_ref[0, 0])[None, :]

  return pl.pallas_call(
      body,
      grid=(N // R,),
      in_specs=[
          pl.BlockSpec((NW, R), lambda i: (0, i)),
          pl.BlockSpec((R, 1), lambda i: (i, 0)),
          pl.BlockSpec((1, 1), lambda i: (0, 0)),
      ],
      out_specs=pl.BlockSpec((1, R), lambda i: (0, i)),
      out_shape=jax.ShapeDtypeStruct((1, N), jnp.float32),
  )(a2p, nin, b2)


def kernel(x, edge_index, W1, b1, W2, b2):
  src = edge_index[0].astype(jnp.int32)
  dst = edge_index[1].astype(jnp.int32)
  srcb = src.reshape(NB, B)
  dstb = dst.reshape(NB, B)

  doutp, dinp = _sc_degrees(src, dst)
  y, nout, nin = _tc_mm1(doutp, dinp, x, W1)
  aggp = _sc_segsum_rows(y, srcb, dstb)
  z = _tc_mm2(aggp, nin, nout, b1.reshape(1, D), W2.reshape(1, D))
  a2p = _sc_segsum_scalar(z.reshape(N), src, dst)
  out = _tc_out(a2p, nin, b2.reshape(1, 1))
  return out.reshape(N, 1)


# speedup vs baseline: 7.4872x; 7.4872x over previous
import dataclasses

import jax
import jax.numpy as jnp
from jax import lax
from jax.experimental import pallas as pl
from jax.experimental.pallas import tpu as pltpu
from jax.experimental.pallas import tpu_sc as plsc

N = 10000
E = 320000
D = 128

NC = 2
NS = 16
NW = NC * NS
LANES = 16

EPW = E // NW
B = 512
NB = E // B
GMAX = (NB + NS - 1) // NS
NHALF = 5000
NR = 5120
DUMP = NHALF
CH = NR // NS

_vec_mesh = plsc.VectorSubcoreMesh(
    core_axis_name="c", subcore_axis_name="s", num_cores=NC, num_subcores=NS
)

_sc_params = pltpu.CompilerParams()
if "needs_layout_passes" in pltpu.CompilerParams.__dataclass_fields__:
  _sc_params = dataclasses.replace(_sc_params, needs_layout_passes=False)


def _sc_degrees(src3, dst3):
  @pl.kernel(
      out_type=(
          jax.ShapeDtypeStruct((NW, 1, N), jnp.float32),
          jax.ShapeDtypeStruct((NW, 1, N), jnp.float32),
      ),
      mesh=_vec_mesh,
      scratch_types=[
          pltpu.VMEM((1, EPW), jnp.int32),
          pltpu.VMEM((1, EPW), jnp.int32),
          pltpu.VMEM((1, N), jnp.float32),
          pltpu.VMEM((1, N), jnp.float32),
      ],
      compiler_params=_sc_params,
  )
  def deg_kernel(src_hbm, dst_hbm, dout_hbm, din_hbm, sv, dv, aout, ain):
    cid = lax.axis_index("c")
    sid = lax.axis_index("s")
    wid = sid * NC + cid

    pltpu.sync_copy(src_hbm.at[wid], sv)
    pltpu.sync_copy(dst_hbm.at[wid], dv)

    zeros = jnp.zeros((LANES,), jnp.float32)
    ones = jnp.ones((LANES,), jnp.float32)

    @pl.loop(0, N, step=LANES)
    def _(i):
      aout[0, pl.ds(i, LANES)] = zeros
      ain[0, pl.ds(i, LANES)] = zeros

    @pl.loop(0, EPW, step=LANES)
    def _(i):
      s = sv[0, pl.ds(i, LANES)]
      d = dv[0, pl.ds(i, LANES)]
      plsc.addupdate_scatter(aout.at[0], [s], ones)
      plsc.addupdate_scatter(ain.at[0], [d], ones)

    pltpu.sync_copy(aout, dout_hbm.at[wid])
    pltpu.sync_copy(ain, din_hbm.at[wid])

  return deg_kernel(src3, dst3)


def _sc_segsum_rows(y, srcb3, dstb3):
  @pl.kernel(
      out_type=jax.ShapeDtypeStruct((NC, NR, D), jnp.float32),
      mesh=_vec_mesh,
      scratch_types=[
          pltpu.VMEM((1, B), jnp.int32),
          pltpu.VMEM((1, B), jnp.int32),
          pltpu.VMEM((1, B), jnp.int32),
          pltpu.VMEM((B, D), jnp.float32),
          pltpu.VMEM_SHARED((NR, D), jnp.float32),
      ],
      compiler_params=_sc_params,
  )
  def seg_kernel(y_hbm, srcb_hbm, dstb_hbm, zero_hbm, out_hbm,
                 si, di, di2, rows, acc):
    cid = lax.axis_index("c")
    sid = lax.axis_index("s")
    base = cid * NHALF

    roff = pl.multiple_of(sid * CH, 8)
    pltpu.sync_copy(zero_hbm.at[pl.ds(roff, CH)], acc.at[pl.ds(roff, CH)])
    plsc.subcore_barrier()

    @pl.loop(0, GMAX)
    def _(g):
      b = g * NS + sid

      @pl.when(b < NB)
      def _():
        pltpu.sync_copy(srcb_hbm.at[b], si)
        pltpu.sync_copy(dstb_hbm.at[b], di)

        @pl.loop(0, B, step=LANES)
        def _(j):
          d = di[0, pl.ds(j, LANES)] - base
          ok = (d >= 0) & (d < NHALF)
          di2[0, pl.ds(j, LANES)] = jnp.where(ok, d, DUMP)

        pltpu.sync_copy(y_hbm.at[si.at[0]], rows)
        pltpu.sync_copy(rows, acc.at[di2.at[0]], add=True)

    plsc.subcore_barrier()
    pltpu.sync_copy(acc.at[pl.ds(roff, CH)],
                    out_hbm.at[cid].at[pl.ds(roff, CH)])

  zero = jnp.zeros((NR, D), jnp.float32)
  return seg_kernel(y, srcb3, dstb3, zero)


def _sc_segsum_scalar(z, src3, dst3):
  @pl.kernel(
      out_type=jax.ShapeDtypeStruct((NW, 1, N), jnp.float32),
      mesh=_vec_mesh,
      scratch_types=[
          pltpu.VMEM((1, N), jnp.float32),
          pltpu.VMEM((1, EPW), jnp.int32),
          pltpu.VMEM((1, EPW), jnp.int32),
          pltpu.VMEM((1, N), jnp.float32),
      ],
      compiler_params=_sc_params,
  )
  def seg2_kernel(z_hbm, src_hbm, dst_hbm, out_hbm, zv, sv, dv, acc):
    cid = lax.axis_index("c")
    sid = lax.axis_index("s")
    wid = sid * NC + cid

    pltpu.sync_copy(z_hbm, zv)
    pltpu.sync_copy(src_hbm.at[wid], sv)
    pltpu.sync_copy(dst_hbm.at[wid], dv)

    zeros = jnp.zeros((LANES,), jnp.float32)

    @pl.loop(0, N, step=LANES)
    def _(i):
      acc[0, pl.ds(i, LANES)] = zeros

    @pl.loop(0, EPW, step=LANES)
    def _(i):
      s = sv[0, pl.ds(i, LANES)]
      d = dv[0, pl.ds(i, LANES)]
      vals = plsc.load_gather(zv.at[0], [s])
      plsc.addupdate_scatter(acc.at[0], [d], vals)

    pltpu.sync_copy(acc, out_hbm.at[wid])

  return seg2_kernel(z, src3, dst3)


R = 1000


def _tc_norms(doutp, dinp):
  def body(doutp_ref, dinp_ref, no_ref, ni_ref, nir_ref):
    no = lax.rsqrt(jnp.clip(jnp.sum(doutp_ref[...], axis=0), 1.0, None))
    ni = lax.rsqrt(jnp.clip(jnp.sum(dinp_ref[...], axis=0), 1.0, None))
    nir_ref[...] = ni[None, :]
    no_ref[...] = no[:, None]
    ni_ref[...] = ni[:, None]

  return pl.pallas_call(
      body,
      in_specs=[
          pl.BlockSpec((NW, N), lambda: (0, 0)),
          pl.BlockSpec((NW, N), lambda: (0, 0)),
      ],
      out_specs=[
          pl.BlockSpec((N, 1), lambda: (0, 0)),
          pl.BlockSpec((N, 1), lambda: (0, 0)),
          pl.BlockSpec((1, N), lambda: (0, 0)),
      ],
      out_shape=[
          jax.ShapeDtypeStruct((N, 1), jnp.float32),
          jax.ShapeDtypeStruct((N, 1), jnp.float32),
          jax.ShapeDtypeStruct((1, N), jnp.float32),
      ],
  )(doutp, dinp)


def _tc_mm1(x, w1, nout):
  def body(x_ref, w1_ref, no_ref, y_ref):
    y_ref[...] = (
        jnp.dot(x_ref[...], w1_ref[...], preferred_element_type=jnp.float32)
        * no_ref[...])

  return pl.pallas_call(
      body,
      grid=(N // R,),
      in_specs=[
          pl.BlockSpec((R, D), lambda i: (i, 0)),
          pl.BlockSpec((D, D), lambda i: (0, 0)),
          pl.BlockSpec((R, 1), lambda i: (i, 0)),
      ],
      out_specs=pl.BlockSpec((R, D), lambda i: (i, 0)),
      out_shape=jax.ShapeDtypeStruct((N, D), jnp.float32),
  )(x, w1, nout)


def _tc_mm2(aggp, nin, nout, b1, w2t):
  def body(agg_ref, ni_ref, no_ref, b1_ref, w2_ref, z_ref):
    h = jnp.maximum(agg_ref[...] * ni_ref[...] + b1_ref[...], 0.0)
    z_ref[...] = jnp.sum(h * w2_ref[...], axis=1, keepdims=True) * no_ref[...]

  return pl.pallas_call(
      body,
      grid=(N // R,),
      in_specs=[
          pl.BlockSpec((R, D), lambda i: (i, 0)),
          pl.BlockSpec((R, 1), lambda i: (i, 0)),
          pl.BlockSpec((R, 1), lambda i: (i, 0)),
          pl.BlockSpec((1, D), lambda i: (0, 0)),
          pl.BlockSpec((1, D), lambda i: (0, 0)),
      ],
      out_specs=pl.BlockSpec((R, 1), lambda i: (i, 0)),
      out_shape=jax.ShapeDtypeStruct((N, 1), jnp.float32),
  )(aggp, nin, nout, b1, w2t)


def _tc_out(a2p, nin_row, b2):
  def body(a2p_ref, ni_ref, b2_ref, o_ref):
    s = jnp.sum(a2p_ref[...], axis=0, keepdims=True)
    o_ref[...] = jax.nn.sigmoid(s * ni_ref[...] + b2_ref[0, 0])

  return pl.pallas_call(
      body,
      in_specs=[
          pl.BlockSpec((NW, N), lambda: (0, 0)),
          pl.BlockSpec((1, N), lambda: (0, 0)),
          pl.BlockSpec((1, 1), lambda: (0, 0)),
      ],
      out_specs=pl.BlockSpec((1, N), lambda: (0, 0)),
      out_shape=jax.ShapeDtypeStruct((1, N), jnp.float32),
  )(a2p, nin_row, b2)


def kernel(x, edge_index, W1, b1, W2, b2):
  src = edge_index[0].astype(jnp.int32)
  dst = edge_index[1].astype(jnp.int32)
  srcb3 = src.reshape(NB, 1, B)
  dstb3 = dst.reshape(NB, 1, B)
  src3 = src.reshape(NW, 1, EPW)
  dst3 = dst.reshape(NW, 1, EPW)

  doutp, dinp = _sc_degrees(src3, dst3)
  nout, nin, nin_row = _tc_norms(doutp.reshape(NW, N), dinp.reshape(NW, N))
  y = _tc_mm1(x, W1, nout)
  aggp = _sc_segsum_rows(y, srcb3, dstb3)
  agg = jnp.concatenate([aggp[0, :NHALF], aggp[1, :NHALF]], axis=0)
  z = _tc_mm2(agg, nin, nout, b1.reshape(1, D), W2.reshape(1, D))
  a2p = _sc_segsum_scalar(z.reshape(1, N), src3, dst3)
  out = _tc_out(a2p.reshape(NW, N), nin_row, b2.reshape(1, 1))
  return out.reshape(N, 1)


# speedup vs baseline: 9.3600x; 1.2501x over previous
import dataclasses

import jax
import jax.numpy as jnp
from jax import lax
from jax.experimental import pallas as pl
from jax.experimental.pallas import tpu as pltpu
from jax.experimental.pallas import tpu_sc as plsc

N = 10000
E = 320000
D = 128

NC = 2
NS = 16
NW = NC * NS
LANES = 16

EPW = E // NW
B = 200
NB = E // B
GMAX = NB // NS
NHALF = 5000
NR = 5120
DUMP = NHALF
CH = NR // NS

_vec_mesh = plsc.VectorSubcoreMesh(
    core_axis_name="c", subcore_axis_name="s", num_cores=NC, num_subcores=NS
)

_sc_params = pltpu.CompilerParams()
if "needs_layout_passes" in pltpu.CompilerParams.__dataclass_fields__:
  _sc_params = dataclasses.replace(_sc_params, needs_layout_passes=False)


def _sc_degrees(src3, dst3):
  @pl.kernel(
      out_type=(
          jax.ShapeDtypeStruct((NW, 1, N), jnp.float32),
          jax.ShapeDtypeStruct((NW, 1, N), jnp.float32),
      ),
      mesh=_vec_mesh,
      scratch_types=[
          pltpu.VMEM((1, EPW), jnp.int32),
          pltpu.VMEM((1, EPW), jnp.int32),
          pltpu.VMEM((1, N), jnp.float32),
          pltpu.VMEM((1, N), jnp.float32),
      ],
      compiler_params=_sc_params,
  )
  def deg_kernel(src_hbm, dst_hbm, dout_hbm, din_hbm, sv, dv, aout, ain):
    cid = lax.axis_index("c")
    sid = lax.axis_index("s")
    wid = sid * NC + cid

    pltpu.sync_copy(src_hbm.at[wid], sv)
    pltpu.sync_copy(dst_hbm.at[wid], dv)

    zeros = jnp.zeros((LANES,), jnp.float32)
    ones = jnp.ones((LANES,), jnp.float32)

    @pl.loop(0, N, step=LANES)
    def _(i):
      aout[0, pl.ds(i, LANES)] = zeros
      ain[0, pl.ds(i, LANES)] = zeros

    @pl.loop(0, EPW, step=LANES)
    def _(i):
      s = sv[0, pl.ds(i, LANES)]
      d = dv[0, pl.ds(i, LANES)]
      plsc.addupdate_scatter(aout.at[0], [s], ones)
      plsc.addupdate_scatter(ain.at[0], [d], ones)

    pltpu.sync_copy(aout, dout_hbm.at[wid])
    pltpu.sync_copy(ain, din_hbm.at[wid])

  return deg_kernel(src3, dst3)


def _sc_segsum_rows(y, srcb3, dstb3):
  @pl.kernel(
      out_type=jax.ShapeDtypeStruct((NC, NR, D), jnp.float32),
      mesh=_vec_mesh,
      scratch_types=[
          pltpu.VMEM((1, B), jnp.int32),
          pltpu.VMEM((1, B), jnp.int32),
          pltpu.VMEM((1, B), jnp.int32),
          pltpu.VMEM((1, B), jnp.int32),
          pltpu.VMEM((1, B), jnp.int32),
          pltpu.VMEM((1, B), jnp.int32),
          pltpu.VMEM((B, D), jnp.float32),
          pltpu.VMEM((B, D), jnp.float32),
          pltpu.VMEM_SHARED((NR, D), jnp.float32),
          pltpu.SemaphoreType.DMA,
          pltpu.SemaphoreType.DMA,
      ],
      compiler_params=_sc_params,
  )
  def seg_kernel(y_hbm, srcb_hbm, dstb_hbm, zero_hbm, out_hbm,
                 si0, di0, dr0, si1, di1, dr1, rows0, rows1, acc,
                 sem0, sem1):
    cid = lax.axis_index("c")
    sid = lax.axis_index("s")
    base = cid * NHALF

    roff = pl.multiple_of(sid * CH, 8)
    pltpu.sync_copy(zero_hbm.at[pl.ds(roff, CH)], acc.at[pl.ds(roff, CH)])
    plsc.subcore_barrier()

    def load_idx(b, si, di, dr):
      pltpu.sync_copy(srcb_hbm.at[b], si)
      pltpu.sync_copy(dstb_hbm.at[b], di)

      @pl.loop(0, B, step=LANES)
      def _(j):
        d = di[0, pl.ds(j, LANES)] - base
        ok = (d >= 0) & (d < NHALF)
        dr[0, pl.ds(j, LANES)] = jnp.where(ok, d, DUMP)

    def g_start(si, rows, sem):
      pltpu.async_copy(y_hbm.at[si.at[0]], rows, sem)

    def g_wait(si, rows, sem):
      pltpu.make_async_copy(y_hbm.at[si.at[0]], rows, sem).wait()

    load_idx(sid, si0, di0, dr0)
    g_start(si0, rows0, sem0)

    @pl.loop(0, GMAX, step=2)
    def _(k):
      b1 = (k + 1) * NS + sid
      load_idx(b1, si1, di1, dr1)
      g_start(si1, rows1, sem1)
      g_wait(si0, rows0, sem0)
      pltpu.sync_copy(rows0, acc.at[dr0.at[0]], add=True)

      @pl.when(k + 2 < GMAX)
      def _():
        b2 = (k + 2) * NS + sid
        load_idx(b2, si0, di0, dr0)
        g_start(si0, rows0, sem0)

      g_wait(si1, rows1, sem1)
      pltpu.sync_copy(rows1, acc.at[dr1.at[0]], add=True)

    plsc.subcore_barrier()
    pltpu.sync_copy(acc.at[pl.ds(roff, CH)],
                    out_hbm.at[cid].at[pl.ds(roff, CH)])

  zero = jnp.zeros((NR, D), jnp.float32)
  return seg_kernel(y, srcb3, dstb3, zero)


def _sc_segsum_scalar(z, src3, dst3):
  @pl.kernel(
      out_type=jax.ShapeDtypeStruct((NW, 1, N), jnp.float32),
      mesh=_vec_mesh,
      scratch_types=[
          pltpu.VMEM((1, N), jnp.float32),
          pltpu.VMEM((1, EPW), jnp.int32),
          pltpu.VMEM((1, EPW), jnp.int32),
          pltpu.VMEM((1, N), jnp.float32),
      ],
      compiler_params=_sc_params,
  )
  def seg2_kernel(z_hbm, src_hbm, dst_hbm, out_hbm, zv, sv, dv, acc):
    cid = lax.axis_index("c")
    sid = lax.axis_index("s")
    wid = sid * NC + cid

    pltpu.sync_copy(z_hbm, zv)
    pltpu.sync_copy(src_hbm.at[wid], sv)
    pltpu.sync_copy(dst_hbm.at[wid], dv)

    zeros = jnp.zeros((LANES,), jnp.float32)

    @pl.loop(0, N, step=LANES)
    def _(i):
      acc[0, pl.ds(i, LANES)] = zeros

    @pl.loop(0, EPW, step=LANES)
    def _(i):
      s = sv[0, pl.ds(i, LANES)]
      d = dv[0, pl.ds(i, LANES)]
      vals = plsc.load_gather(zv.at[0], [s])
      plsc.addupdate_scatter(acc.at[0], [d], vals)

    pltpu.sync_copy(acc, out_hbm.at[wid])

  return seg2_kernel(z, src3, dst3)


R = 1000


def _tc_norms(doutp, dinp):
  def body(doutp_ref, dinp_ref, no_ref, ni_ref, nir_ref):
    no = lax.rsqrt(jnp.clip(jnp.sum(doutp_ref[...], axis=0), 1.0, None))
    ni = lax.rsqrt(jnp.clip(jnp.sum(dinp_ref[...], axis=0), 1.0, None))
    nir_ref[...] = ni[None, :]
    no_ref[...] = no[:, None]
    ni_ref[...] = ni[:, None]

  return pl.pallas_call(
      body,
      in_specs=[
          pl.BlockSpec((NW, N), lambda: (0, 0)),
          pl.BlockSpec((NW, N), lambda: (0, 0)),
      ],
      out_specs=[
          pl.BlockSpec((N, 1), lambda: (0, 0)),
          pl.BlockSpec((N, 1), lambda: (0, 0)),
          pl.BlockSpec((1, N), lambda: (0, 0)),
      ],
      out_shape=[
          jax.ShapeDtypeStruct((N, 1), jnp.float32),
          jax.ShapeDtypeStruct((N, 1), jnp.float32),
          jax.ShapeDtypeStruct((1, N), jnp.float32),
      ],
  )(doutp, dinp)


def _tc_mm1(x, w1, nout):
  def body(x_ref, w1_ref, no_ref, y_ref):
    y_ref[...] = (
        jnp.dot(x_ref[...], w1_ref[...], preferred_element_type=jnp.float32)
        * no_ref[...])

  return pl.pallas_call(
      body,
      grid=(N // R,),
      in_specs=[
          pl.BlockSpec((R, D), lambda i: (i, 0)),
          pl.BlockSpec((D, D), lambda i: (0, 0)),
          pl.BlockSpec((R, 1), lambda i: (i, 0)),
      ],
      out_specs=pl.BlockSpec((R, D), lambda i: (i, 0)),
      out_shape=jax.ShapeDtypeStruct((N, D), jnp.float32),
  )(x, w1, nout)


R2 = 1000


def _tc_mm2(aggp, nin, nout, b1, w2t):
  bpc = NHALF // R2

  def body(agg_ref, ni_ref, no_ref, b1_ref, w2_ref, z_ref):
    h = jnp.maximum(agg_ref[0] * ni_ref[...] + b1_ref[...], 0.0)
    z_ref[...] = jnp.sum(h * w2_ref[...], axis=1, keepdims=True) * no_ref[...]

  return pl.pallas_call(
      body,
      grid=(N // R2,),
      in_specs=[
          pl.BlockSpec((1, R2, D), lambda i: (i // bpc, i % bpc, 0)),
          pl.BlockSpec((R2, 1), lambda i: (i, 0)),
          pl.BlockSpec((R2, 1), lambda i: (i, 0)),
          pl.BlockSpec((1, D), lambda i: (0, 0)),
          pl.BlockSpec((1, D), lambda i: (0, 0)),
      ],
      out_specs=pl.BlockSpec((R2, 1), lambda i: (i, 0)),
      out_shape=jax.ShapeDtypeStruct((N, 1), jnp.float32),
  )(aggp, nin, nout, b1, w2t)


def _tc_out(a2p, nin_row, b2):
  def body(a2p_ref, ni_ref, b2_ref, o_ref):
    s = jnp.sum(a2p_ref[...], axis=0, keepdims=True)
    o_ref[...] = jax.nn.sigmoid(s * ni_ref[...] + b2_ref[0, 0])

  return pl.pallas_call(
      body,
      in_specs=[
          pl.BlockSpec((NW, N), lambda: (0, 0)),
          pl.BlockSpec((1, N), lambda: (0, 0)),
          pl.BlockSpec((1, 1), lambda: (0, 0)),
      ],
      out_specs=pl.BlockSpec((1, N), lambda: (0, 0)),
      out_shape=jax.ShapeDtypeStruct((1, N), jnp.float32),
  )(a2p, nin_row, b2)


def kernel(x, edge_index, W1, b1, W2, b2):
  src = edge_index[0].astype(jnp.int32)
  dst = edge_index[1].astype(jnp.int32)
  srcb3 = src.reshape(NB, 1, B)
  dstb3 = dst.reshape(NB, 1, B)
  src3 = src.reshape(NW, 1, EPW)
  dst3 = dst.reshape(NW, 1, EPW)

  doutp, dinp = _sc_degrees(src3, dst3)
  nout, nin, nin_row = _tc_norms(doutp.reshape(NW, N), dinp.reshape(NW, N))
  y = _tc_mm1(x, W1, nout)
  aggp = _sc_segsum_rows(y, srcb3, dstb3)
  z = _tc_mm2(aggp, nin, nout, b1.reshape(1, D), W2.reshape(1, D))
  a2p = _sc_segsum_scalar(z.reshape(1, N), src3, dst3)
  out = _tc_out(a2p.reshape(NW, N), nin_row, b2.reshape(1, 1))
  return out.reshape(N, 1)
